# baseline (speedup 1.0000x reference)
import functools

import jax
import jax.numpy as jnp
from jax.experimental import pallas as pl
from jax.experimental.pallas import tpu as pltpu

F32 = jnp.float32
BF16 = jnp.bfloat16
_EPS = 1e-5
_H = 16
_KMAX = 64


def _ln(x, g, b):
    mu = jnp.mean(x, axis=-1, keepdims=True)
    var = jnp.mean((x - mu) ** 2, axis=-1, keepdims=True)
    return (x - mu) * jax.lax.rsqrt(var + _EPS) * g + b


def _mmt(a, b):
    return jax.lax.dot_general(a, b, (((1,), (1,)), ((), ())),
                               preferred_element_type=F32)


def _mmt_hi(a, b):
    return jax.lax.dot_general(a, b, (((1,), (1,)), ((), ())),
                               preferred_element_type=F32,
                               precision=jax.lax.Precision.HIGHEST)


def _mm(a, b):
    return jax.lax.dot_general(a, b, (((1,), (0,)), ((), ())),
                               preferred_element_type=F32)


def _summary_body(h_ref, g_ref, b_ref, out_ref, *, inv_t):
    t = pl.program_id(1)
    xn = _ln(h_ref[0], g_ref[...], b_ref[...])

    @pl.when(t == 0)
    def _():
        out_ref[...] = jnp.zeros_like(out_ref)

    out_ref[...] += jnp.sum(xn, axis=0, keepdims=True)[None]

    @pl.when(t == pl.num_programs(1) - 1)
    def _():
        out_ref[...] *= inv_t


def _select_body(sum_ref, wf_ref, bf_ref, act_ref, aw_ref, mem_ref, idx_ref,
                 sel_ref, *, k_sel):
    fq = _mmt_hi(sum_ref[...], wf_ref[...]) + bf_ref[...]
    sel_ref[...] = _mmt_hi(fq, mem_ref[...]) + aw_ref[0, 0] * act_ref[...]
    bsz, n = sel_ref.shape
    iota_n = jax.lax.broadcasted_iota(jnp.int32, (bsz, n), 1)
    iota_k = jax.lax.broadcasted_iota(jnp.int32, (bsz, k_sel), 1)

    def body(kk, idxacc):
        sel = sel_ref[...]
        m = jnp.max(sel, axis=1, keepdims=True)
        cand = jnp.where(sel >= m, iota_n, jnp.int32(n))
        j = jnp.min(cand, axis=1, keepdims=True)
        sel_ref[...] = jnp.where(iota_n == j, -jnp.inf, sel)
        return jnp.where(iota_k == kk, j, idxacc)

    idx_ref[...] = jax.lax.fori_loop(
        0, k_sel, body, jnp.zeros((bsz, k_sel), jnp.int32))


def _gather_kv_body(idx_ref, mem_ref, wk_ref, bk_ref, wv_ref, bv_ref,
                    k_ref, v_ref, tm_ref):
    bsz, k_sel = idx_ref.shape

    def body(j, _):
        r = idx_ref[j // k_sel, j % k_sel]
        tm_ref[pl.ds(j, 1), :] = mem_ref[pl.ds(r, 1), :]
        return 0

    jax.lax.fori_loop(0, bsz * k_sel, body, 0)
    tm = tm_ref[...].astype(BF16)
    k_ref[...] = (_mmt(tm, wk_ref[...]) + bk_ref[...]).astype(BF16)
    v_ref[...] = (_mmt(tm, wv_ref[...]) + bv_ref[...]).astype(BF16)


def _attn_body(h_ref, k_ref, v_ref, wq_ref, bq_ref, wo_ref, bo_ref,
               g_ref, b_ref, gl_ref, hu_ref, asum_ref, o_scr, *, scale, heads):
    t = pl.program_id(1)
    x = h_ref[0]
    xn = _ln(x, g_ref[...], b_ref[...])
    q = _mmt(xn.astype(BF16), wq_ref[...]) + bq_ref[...]
    qb = (q * scale).astype(BF16)
    k_sel = k_ref.shape[0]
    dh = x.shape[1] // heads
    acc = jnp.zeros((1, k_sel), F32)
    for hh in range(heads):
        sl = slice(hh * dh, (hh + 1) * dh)
        s = _mmt(qb[:, sl], k_ref[:, sl])
        m = jnp.max(s, axis=1, keepdims=True)
        e = jnp.exp(s - m)
        p = e / jnp.sum(e, axis=1, keepdims=True)
        acc = acc + jnp.sum(p, axis=0, keepdims=True)
        o_scr[:, sl] = _mm(p.astype(BF16), v_ref[:, sl])
    out = _mmt(o_scr[...].astype(BF16), wo_ref[...]) + bo_ref[...]
    gate = 1.0 / (1.0 + jnp.exp(-gl_ref[0, 0]))
    hu_ref[0] = x + gate * out

    @pl.when(t == 0)
    def _():
        asum_ref[...] = jnp.zeros_like(asum_ref)

    asum_ref[...] += acc[None]


def _scatter_body(idx_ref, asum_ref, fa_ref, *, inv_ht):
    b = pl.program_id(0)
    n = fa_ref.shape[2]
    k_sel = idx_ref.shape[1]
    iota = jax.lax.broadcasted_iota(jnp.int32, (1, n), 1)
    fa = jnp.zeros((1, n), F32)
    for kk in range(k_sel):
        fa = fa + jnp.where(iota == idx_ref[b, kk],
                            asum_ref[b, kk] * inv_ht, 0.0)
    fa_ref[...] = fa[None]


def kernel(h, memory, activations, Wq, bq, Wk, bk, Wv, bv, Wo, bo,
           ln_g, ln_b, Wf, bf, activation_weight, gate_logit):
    B, T, d = h.shape
    N = memory.shape[0]
    K = min(_KMAX, N)
    H = _H
    TT = min(512, T)
    nT = T // TT

    g2 = ln_g.reshape(1, d)
    b2 = ln_b.reshape(1, d)
    bq2 = bq.reshape(1, d)
    bo2 = bo.reshape(1, d)
    bf2 = bf.reshape(1, d)
    bk2 = bk.reshape(1, d)
    bv2 = bv.reshape(1, d)
    aw2 = activation_weight.reshape(1, 1)
    gl2 = gate_logit.reshape(1, 1)
    wq_b = Wq.astype(BF16)
    wo_b = Wo.astype(BF16)
    wk_b = Wk.astype(BF16)
    wv_b = Wv.astype(BF16)

    summary = pl.pallas_call(
        functools.partial(_summary_body, inv_t=1.0 / T),
        grid=(B, nT),
        in_specs=[
            pl.BlockSpec((1, TT, d), lambda b_, t_: (b_, t_, 0)),
            pl.BlockSpec((1, d), lambda b_, t_: (0, 0)),
            pl.BlockSpec((1, d), lambda b_, t_: (0, 0)),
        ],
        out_specs=pl.BlockSpec((1, 1, d), lambda b_, t_: (b_, 0, 0)),
        out_shape=jax.ShapeDtypeStruct((B, 1, d), F32),
    )(h, g2, b2)
    summary = summary.reshape(B, d)

    idx = pl.pallas_call(
        functools.partial(_select_body, k_sel=K),
        in_specs=[
            pl.BlockSpec(memory_space=pltpu.VMEM),
            pl.BlockSpec(memory_space=pltpu.VMEM),
            pl.BlockSpec(memory_space=pltpu.VMEM),
            pl.BlockSpec(memory_space=pltpu.VMEM),
            pl.BlockSpec(memory_space=pltpu.SMEM),
            pl.BlockSpec(memory_space=pltpu.VMEM),
        ],
        out_specs=pl.BlockSpec(memory_space=pltpu.VMEM),
        out_shape=jax.ShapeDtypeStruct((B, K), jnp.int32),
        scratch_shapes=[pltpu.VMEM((B, N), F32)],
    )(summary, Wf, bf2, activations, aw2, memory)

    k2, v2 = pl.pallas_call(
        _gather_kv_body,
        in_specs=[
            pl.BlockSpec(memory_space=pltpu.SMEM),
            pl.BlockSpec(memory_space=pltpu.VMEM),
            pl.BlockSpec(memory_space=pltpu.VMEM),
            pl.BlockSpec(memory_space=pltpu.VMEM),
            pl.BlockSpec(memory_space=pltpu.VMEM),
            pl.BlockSpec(memory_space=pltpu.VMEM),
        ],
        out_specs=[
            pl.BlockSpec(memory_space=pltpu.VMEM),
            pl.BlockSpec(memory_space=pltpu.VMEM),
        ],
        out_shape=[
            jax.ShapeDtypeStruct((B * K, d), BF16),
            jax.ShapeDtypeStruct((B * K, d), BF16),
        ],
        scratch_shapes=[pltpu.VMEM((B * K, d), F32)],
    )(idx, memory, wk_b, bk2, wv_b, bv2)

    hu, asum = pl.pallas_call(
        functools.partial(_attn_body, scale=1.0 / (64 ** 0.5) if d // H == 64
                          else (d // H) ** -0.5, heads=H),
        grid=(B, nT),
        in_specs=[
            pl.BlockSpec((1, TT, d), lambda b_, t_: (b_, t_, 0)),
            pl.BlockSpec((K, d), lambda b_, t_: (b_, 0)),
            pl.BlockSpec((K, d), lambda b_, t_: (b_, 0)),
            pl.BlockSpec((d, d), lambda b_, t_: (0, 0)),
            pl.BlockSpec((1, d), lambda b_, t_: (0, 0)),
            pl.BlockSpec((d, d), lambda b_, t_: (0, 0)),
            pl.BlockSpec((1, d), lambda b_, t_: (0, 0)),
            pl.BlockSpec((1, d), lambda b_, t_: (0, 0)),
            pl.BlockSpec((1, d), lambda b_, t_: (0, 0)),
            pl.BlockSpec(memory_space=pltpu.SMEM),
        ],
        out_specs=[
            pl.BlockSpec((1, TT, d), lambda b_, t_: (b_, t_, 0)),
            pl.BlockSpec((1, 1, K), lambda b_, t_: (b_, 0, 0)),
        ],
        out_shape=[
            jax.ShapeDtypeStruct((B, T, d), F32),
            jax.ShapeDtypeStruct((B, 1, K), F32),
        ],
        scratch_shapes=[pltpu.VMEM((TT, d), F32)],
    )(h, k2, v2, wq_b, bq2, wo_b, bo2, g2, b2, gl2)
    asum = asum.reshape(B, K)

    fa = pl.pallas_call(
        functools.partial(_scatter_body, inv_ht=1.0 / (H * T)),
        grid=(B,),
        in_specs=[
            pl.BlockSpec(memory_space=pltpu.SMEM),
            pl.BlockSpec(memory_space=pltpu.SMEM),
        ],
        out_specs=pl.BlockSpec((1, 1, N), lambda b_: (b_, 0, 0)),
        out_shape=jax.ShapeDtypeStruct((B, 1, N), F32),
    )(idx, asum)

    return hu, fa.reshape(B, N)


# speedup vs baseline: 1.2147x; 1.2147x over previous
import functools

import jax
import jax.numpy as jnp
from jax.experimental import pallas as pl
from jax.experimental.pallas import tpu as pltpu

F32 = jnp.float32
BF16 = jnp.bfloat16
_EPS = 1e-5
_H = 16
_KMAX = 64


def _ln(x, g, b):
    mu = jnp.mean(x, axis=-1, keepdims=True)
    var = jnp.mean((x - mu) ** 2, axis=-1, keepdims=True)
    return (x - mu) * jax.lax.rsqrt(var + _EPS) * g + b


def _mmt(a, b):
    return jax.lax.dot_general(a, b, (((1,), (1,)), ((), ())),
                               preferred_element_type=F32)


def _mmt_hi(a, b):
    return jax.lax.dot_general(a, b, (((1,), (1,)), ((), ())),
                               preferred_element_type=F32,
                               precision=jax.lax.Precision.HIGHEST)


def _mm(a, b):
    return jax.lax.dot_general(a, b, (((1,), (0,)), ((), ())),
                               preferred_element_type=F32)


def _summary_body(h_ref, g_ref, b_ref, out_ref, *, inv_t):
    t = pl.program_id(1)
    xn = _ln(h_ref[0], g_ref[...], b_ref[...])

    @pl.when(t == 0)
    def _():
        out_ref[...] = jnp.zeros_like(out_ref)

    out_ref[...] += jnp.sum(xn, axis=0, keepdims=True)[None]

    @pl.when(t == pl.num_programs(1) - 1)
    def _():
        out_ref[...] *= inv_t


def _select_body(sum_ref, wf_ref, bf_ref, act_ref, aw_ref, mem_ref, idx_ref,
                 sel_ref, *, k_sel):
    fq = _mmt_hi(sum_ref[...], wf_ref[...]) + bf_ref[...]
    sel_ref[...] = _mmt(fq, mem_ref[...]) + aw_ref[0, 0] * act_ref[...]
    bsz, n = sel_ref.shape
    iota_n = jax.lax.broadcasted_iota(jnp.int32, (bsz, n), 1)
    iota_k = jax.lax.broadcasted_iota(jnp.int32, (bsz, k_sel), 1)

    def body(kk, idxacc):
        sel = sel_ref[...]
        m = jnp.max(sel, axis=1, keepdims=True)
        cand = jnp.where(sel >= m, iota_n, jnp.int32(n))
        j = jnp.min(cand, axis=1, keepdims=True)
        sel_ref[...] = jnp.where(iota_n == j, -jnp.inf, sel)
        return jnp.where(iota_k == kk, j, idxacc)

    idx_ref[...] = jax.lax.fori_loop(
        0, k_sel, body, jnp.zeros((bsz, k_sel), jnp.int32))


def _gather_body(idx_ref, mem_ref, tm_ref):
    bsz, k_sel = idx_ref.shape

    def body(j, _):
        r = idx_ref[j // k_sel, j % k_sel]
        tm_ref[pl.ds(j, 1), :] = mem_ref[pl.ds(r, 1), :]
        return 0

    jax.lax.fori_loop(0, bsz * k_sel, body, 0)


def _kv_pack_body(tm_ref, wk_ref, bk_ref, wv_ref, bv_ref, mask_ref,
                  km_ref, vm_ref, *, bsz, heads):
    tm = tm_ref[...].astype(BF16)
    kf = _mmt(tm, wk_ref[...]) + bk_ref[...]
    vf = _mmt(tm, wv_ref[...]) + bv_ref[...]
    k_sel = tm_ref.shape[0] // bsz
    d = tm_ref.shape[1]
    mask = mask_ref[...]
    for b in range(bsz):
        kb = kf[b * k_sel:(b + 1) * k_sel].astype(BF16)
        vb = vf[b * k_sel:(b + 1) * k_sel].astype(BF16)
        ktile = jnp.broadcast_to(kb[None], (heads, k_sel, d)).reshape(d, d)
        vtile = jnp.broadcast_to(vb[None], (heads, k_sel, d)).reshape(d, d)
        km_ref[b] = ktile * mask
        vm_ref[b] = vtile * mask


def _attn_body(h_ref, km_ref, vm_ref, mask_ref, wq_ref, bq_ref, wo_ref,
               bo_ref, g_ref, b_ref, gl_ref, hu_ref, asum_ref,
               *, scale, heads):
    t = pl.program_id(1)
    x = h_ref[0]
    xn = _ln(x, g_ref[...], b_ref[...])
    q = _mmt(xn.astype(BF16), wq_ref[...]) + bq_ref[...]
    qb = (q * scale).astype(BF16)
    d = x.shape[1]
    k_sel = d // heads
    s_all = _mmt(qb, km_ref[0])
    e = jnp.exp(s_all)
    eb = e.astype(BF16)
    den = _mm(eb, mask_ref[...])
    p = e / den
    psum = jnp.sum(p, axis=0, keepdims=True)
    acc = psum[:, 0:k_sel]
    for hh in range(1, heads):
        acc = acc + psum[:, hh * k_sel:(hh + 1) * k_sel]
    o_all = _mm(p.astype(BF16), vm_ref[0])
    out = _mmt(o_all.astype(BF16), wo_ref[...]) + bo_ref[...]
    gate = 1.0 / (1.0 + jnp.exp(-gl_ref[0, 0]))
    hu_ref[0] = x + gate * out

    @pl.when(t == 0)
    def _():
        asum_ref[...] = jnp.zeros_like(asum_ref)

    asum_ref[...] += acc[None]


def _scatter_body(idx_ref, asum_ref, fa_ref, *, inv_ht):
    b = pl.program_id(0)
    n = fa_ref.shape[2]
    k_sel = idx_ref.shape[1]
    iota = jax.lax.broadcasted_iota(jnp.int32, (1, n), 1)
    fa = jnp.zeros((1, n), F32)
    for kk in range(k_sel):
        fa = fa + jnp.where(iota == idx_ref[b, kk],
                            asum_ref[b, kk] * inv_ht, 0.0)
    fa_ref[...] = fa[None]


def kernel(h, memory, activations, Wq, bq, Wk, bk, Wv, bv, Wo, bo,
           ln_g, ln_b, Wf, bf, activation_weight, gate_logit):
    B, T, d = h.shape
    N = memory.shape[0]
    K = min(_KMAX, N)
    H = _H
    TT = min(512, T)
    nT = T // TT

    g2 = ln_g.reshape(1, d)
    b2 = ln_b.reshape(1, d)
    bq2 = bq.reshape(1, d)
    bo2 = bo.reshape(1, d)
    bf2 = bf.reshape(1, d)
    bk2 = bk.reshape(1, d)
    bv2 = bv.reshape(1, d)
    aw2 = activation_weight.reshape(1, 1)
    gl2 = gate_logit.reshape(1, 1)
    wq_b = Wq.astype(BF16)
    wo_b = Wo.astype(BF16)
    wk_b = Wk.astype(BF16)
    wv_b = Wv.astype(BF16)
    ii = jnp.arange(d, dtype=jnp.int32) // (d // H)
    mask_bd = (ii[:, None] == ii[None, :]).astype(BF16)

    summary = pl.pallas_call(
        functools.partial(_summary_body, inv_t=1.0 / T),
        grid=(B, nT),
        in_specs=[
            pl.BlockSpec((1, TT, d), lambda b_, t_: (b_, t_, 0)),
            pl.BlockSpec((1, d), lambda b_, t_: (0, 0)),
            pl.BlockSpec((1, d), lambda b_, t_: (0, 0)),
        ],
        out_specs=pl.BlockSpec((1, 1, d), lambda b_, t_: (b_, 0, 0)),
        out_shape=jax.ShapeDtypeStruct((B, 1, d), F32),
    )(h, g2, b2)
    summary = summary.reshape(B, d)

    idx = pl.pallas_call(
        functools.partial(_select_body, k_sel=K),
        in_specs=[pl.BlockSpec(memory_space=pltpu.VMEM)] * 4
        + [pl.BlockSpec(memory_space=pltpu.SMEM),
           pl.BlockSpec(memory_space=pltpu.VMEM)],
        out_specs=pl.BlockSpec(memory_space=pltpu.VMEM),
        out_shape=jax.ShapeDtypeStruct((B, K), jnp.int32),
        scratch_shapes=[pltpu.VMEM((B, N), F32)],
    )(summary, Wf, bf2, activations, aw2, memory)

    tm = pl.pallas_call(
        _gather_body,
        in_specs=[pl.BlockSpec(memory_space=pltpu.SMEM),
                  pl.BlockSpec(memory_space=pltpu.VMEM)],
        out_specs=pl.BlockSpec(memory_space=pltpu.VMEM),
        out_shape=jax.ShapeDtypeStruct((B * K, d), F32),
    )(idx, memory)

    km, vm = pl.pallas_call(
        functools.partial(_kv_pack_body, bsz=B, heads=H),
        in_specs=[pl.BlockSpec(memory_space=pltpu.VMEM)] * 6,
        out_specs=[pl.BlockSpec(memory_space=pltpu.VMEM)] * 2,
        out_shape=[jax.ShapeDtypeStruct((B, d, d), BF16)] * 2,
    )(tm, wk_b, bk2, wv_b, bv2, mask_bd)

    hu, asum = pl.pallas_call(
        functools.partial(_attn_body, scale=(d // H) ** -0.5, heads=H),
        grid=(B, nT),
        in_specs=[
            pl.BlockSpec((1, TT, d), lambda b_, t_: (b_, t_, 0)),
            pl.BlockSpec((1, d, d), lambda b_, t_: (b_, 0, 0)),
            pl.BlockSpec((1, d, d), lambda b_, t_: (b_, 0, 0)),
            pl.BlockSpec((d, d), lambda b_, t_: (0, 0)),
            pl.BlockSpec((d, d), lambda b_, t_: (0, 0)),
            pl.BlockSpec((1, d), lambda b_, t_: (0, 0)),
            pl.BlockSpec((d, d), lambda b_, t_: (0, 0)),
            pl.BlockSpec((1, d), lambda b_, t_: (0, 0)),
            pl.BlockSpec((1, d), lambda b_, t_: (0, 0)),
            pl.BlockSpec((1, d), lambda b_, t_: (0, 0)),
            pl.BlockSpec(memory_space=pltpu.SMEM),
        ],
        out_specs=[
            pl.BlockSpec((1, TT, d), lambda b_, t_: (b_, t_, 0)),
            pl.BlockSpec((1, 1, K), lambda b_, t_: (b_, 0, 0)),
        ],
        out_shape=[
            jax.ShapeDtypeStruct((B, T, d), F32),
            jax.ShapeDtypeStruct((B, 1, K), F32),
        ],
    )(h, km, vm, mask_bd, wq_b, bq2, wo_b, bo2, g2, b2, gl2)
    asum = asum.reshape(B, K)

    fa = pl.pallas_call(
        functools.partial(_scatter_body, inv_ht=1.0 / (H * T)),
        grid=(B,),
        in_specs=[
            pl.BlockSpec(memory_space=pltpu.SMEM),
            pl.BlockSpec(memory_space=pltpu.SMEM),
        ],
        out_specs=pl.BlockSpec((1, 1, N), lambda b_: (b_, 0, 0)),
        out_shape=jax.ShapeDtypeStruct((B, 1, N), F32),
    )(idx, asum)

    return hu, fa.reshape(B, N)
